# baseline (speedup 1.0000x reference)
import jax
import jax.numpy as jnp
from jax.experimental import pallas as pl

_B, _D, _K = 4096, 64, 8192
_BB = 128


def _body(x_ref, w_ref, dist_ref, imin_ref):
    x = x_ref[...]
    w = w_ref[...]
    cross = jax.lax.dot_general(
        x, w, (((1,), (0,)), ((), ())),
        preferred_element_type=jnp.float32)
    x_sq = jnp.sum(x * x, axis=1, keepdims=True)
    w_sq = jnp.sum(w * w, axis=0, keepdims=True)
    d = x_sq - 2.0 * cross + w_sq
    dist_ref[...] = d
    imin_ref[...] = jnp.argmin(d, axis=1).astype(jnp.int32)[:, None]


def kernel(x, w):
    dist, imin = pl.pallas_call(
        _body,
        grid=(_B // _BB,),
        in_specs=[
            pl.BlockSpec((_BB, _D), lambda b: (b, 0)),
            pl.BlockSpec((_D, _K), lambda b: (0, 0)),
        ],
        out_specs=[
            pl.BlockSpec((_BB, _K), lambda b: (b, 0)),
            pl.BlockSpec((_BB, 1), lambda b: (b, 0)),
        ],
        out_shape=[
            jax.ShapeDtypeStruct((_B, _K), jnp.float32),
            jax.ShapeDtypeStruct((_B, 1), jnp.int32),
        ],
    )(x, w)
    return dist, imin.reshape(_B)


# speedup vs baseline: 1.1634x; 1.1634x over previous
import jax
import jax.numpy as jnp
from jax.experimental import pallas as pl
from jax.experimental.pallas import tpu as pltpu

_B, _D, _K = 4096, 64, 8192
_BB = 512


def _body(x_ref, w_ref, dist_ref, imin_ref, wsq_ref):
    @pl.when(pl.program_id(0) == 0)
    def _():
        w0 = w_ref[...]
        wsq_ref[...] = jnp.sum(w0 * w0, axis=0, keepdims=True)

    x = x_ref[...]
    xm2 = x * (-2.0)
    cross = jax.lax.dot_general(
        xm2, w_ref[...], (((1,), (0,)), ((), ())),
        preferred_element_type=jnp.float32)
    x_sq = jnp.sum(x * x, axis=1, keepdims=True)
    d = (x_sq + cross) + wsq_ref[...]
    dist_ref[...] = d
    imin_ref[...] = jnp.argmin(d, axis=1).astype(jnp.int32)[:, None]


def kernel(x, w):
    dist, imin = pl.pallas_call(
        _body,
        grid=(_B // _BB,),
        in_specs=[
            pl.BlockSpec((_BB, _D), lambda b: (b, 0)),
            pl.BlockSpec((_D, _K), lambda b: (0, 0)),
        ],
        out_specs=[
            pl.BlockSpec((_BB, _K), lambda b: (b, 0)),
            pl.BlockSpec((_BB, 1), lambda b: (b, 0)),
        ],
        out_shape=[
            jax.ShapeDtypeStruct((_B, _K), jnp.float32),
            jax.ShapeDtypeStruct((_B, 1), jnp.int32),
        ],
        scratch_shapes=[pltpu.VMEM((1, _K), jnp.float32)],
    )(x, w)
    return dist, imin.reshape(_B)


# speedup vs baseline: 1.1944x; 1.0267x over previous
import jax
import jax.numpy as jnp
from jax.experimental import pallas as pl
from jax.experimental.pallas import tpu as pltpu

_B, _D, _K = 4096, 64, 8192
_BB = 256


def _body(x_ref, w_ref, dist_ref, imin_ref, wsq_ref):
    @pl.when(pl.program_id(0) == 0)
    def _():
        w0 = w_ref[...]
        wsq_ref[...] = jnp.sum(w0 * w0, axis=0, keepdims=True)

    x = x_ref[...]
    xm2 = x * (-2.0)
    cross = jax.lax.dot_general(
        xm2, w_ref[...], (((1,), (0,)), ((), ())),
        preferred_element_type=jnp.float32)
    x_sq = jnp.sum(x * x, axis=1, keepdims=True)
    d = (x_sq + cross) + wsq_ref[...]
    dist_ref[...] = d
    imin_ref[...] = jnp.argmin(d, axis=1).astype(jnp.int32)[:, None]


def kernel(x, w):
    dist, imin = pl.pallas_call(
        _body,
        grid=(_B // _BB,),
        in_specs=[
            pl.BlockSpec((_BB, _D), lambda b: (b, 0)),
            pl.BlockSpec((_D, _K), lambda b: (0, 0)),
        ],
        out_specs=[
            pl.BlockSpec((_BB, _K), lambda b: (b, 0)),
            pl.BlockSpec((_BB, 1), lambda b: (b, 0)),
        ],
        out_shape=[
            jax.ShapeDtypeStruct((_B, _K), jnp.float32),
            jax.ShapeDtypeStruct((_B, 1), jnp.int32),
        ],
        scratch_shapes=[pltpu.VMEM((1, _K), jnp.float32)],
    )(x, w)
    return dist, imin.reshape(_B)
